# baseline (speedup 1.0000x reference)
import functools

import jax
import jax.numpy as jnp
from jax import lax
from jax.experimental import pallas as pl
from jax.experimental.pallas import tpu as pltpu
from jax.experimental.pallas import tpu_sc as plsc

D_MODEL = 256
HALF = 128
N_PAT = 64
LANES = 16

B, H, W = 128, 30, 30
N = B * H * W
HW = H * W
NW = 32
PER_W = N // NW
CHUNK = 80
NCHUNK = PER_W // CHUNK


def _body(x_hbm, idx_hbm, sp_hbm, ppe_hbm, out_hbm,
          x_v, sp_v, pt_v, ptidx_v, spidx_v, sem_sp, sem_pt, sem_x):
    wid = lax.axis_index("s") * 2 + lax.axis_index("c")
    base = wid * PER_W
    iota = lax.iota(jnp.int32, LANES)

    def chunk_body(c, carry):
        p0 = base + c * CHUNK
        pltpu.sync_copy(idx_hbm.at[pl.ds(p0, CHUNK)], ptidx_v)
        for k in range(CHUNK // LANES):
            sl = pl.ds(k * LANES, LANES)
            ptidx_v[sl] = lax.bitwise_and(ptidx_v[sl], N_PAT - 1)
            spidx_v[sl] = lax.rem(p0 + k * LANES + iota, HW)
        cp_sp = pltpu.async_copy(sp_hbm.at[spidx_v], sp_v, sem_sp)
        cp_pt = pltpu.async_copy(ppe_hbm.at[ptidx_v], pt_v, sem_pt)
        cp_x = pltpu.async_copy(x_hbm.at[pl.ds(p0, CHUNK)], x_v, sem_x)
        cp_sp.wait()
        cp_pt.wait()
        cp_x.wait()

        def add_body(j, carry2):
            for k in range(HALF // LANES):
                sl = pl.ds(k * LANES, LANES)
                sh = pl.ds(HALF + k * LANES, LANES)
                x_v[j, sl] = x_v[j, sl] + sp_v[j, sl]
                x_v[j, sh] = x_v[j, sh] + pt_v[j, sl]
            return carry2

        lax.fori_loop(0, CHUNK, add_body, 0)
        pltpu.sync_copy(x_v, out_hbm.at[pl.ds(p0, CHUNK)])
        return carry

    lax.fori_loop(0, NCHUNK, chunk_body, 0)


@jax.jit
def kernel(x, pattern_indices, spatial_pe, pattern_pe):
    b, h, w, d = x.shape
    xf = x.reshape(N, D_MODEL)
    idxf = pattern_indices.reshape(N).astype(jnp.int32)
    spf = spatial_pe.reshape(HW, HALF)

    mesh = plsc.VectorSubcoreMesh(core_axis_name="c", subcore_axis_name="s")
    out = pl.kernel(
        _body,
        out_type=jax.ShapeDtypeStruct((N, D_MODEL), jnp.float32),
        mesh=mesh,
        scratch_types=[
            pltpu.VMEM((CHUNK, D_MODEL), jnp.float32),
            pltpu.VMEM((CHUNK, HALF), jnp.float32),
            pltpu.VMEM((CHUNK, HALF), jnp.float32),
            pltpu.VMEM((CHUNK,), jnp.int32),
            pltpu.VMEM((CHUNK,), jnp.int32),
            pltpu.SemaphoreType.DMA,
            pltpu.SemaphoreType.DMA,
            pltpu.SemaphoreType.DMA,
        ],
    )(xf, idxf, spf, pattern_pe)
    return out.reshape(b, h, w, d)


# speedup vs baseline: 1.1770x; 1.1770x over previous
import jax
import jax.numpy as jnp
from jax import lax
from jax.experimental import pallas as pl
from jax.experimental.pallas import tpu as pltpu
from jax.experimental.pallas import tpu_sc as plsc

D_MODEL = 256
HALF = 128
N_PAT = 64
LANES = 16

B, H, W = 128, 30, 30
N = B * H * W
HW = H * W
NW = 32
PER_W = N // NW
CHUNK = 48
NCHUNK = PER_W // CHUNK
NBUF = 3


def _body(x_hbm, idx_hbm, sp_hbm, ppe_hbm, out_hbm,
          x_v, sp_v, pt_v, pti_v, spi_v,
          si0, si1, si2, so0, so1, so2):
    sem_in = (si0, si1, si2)
    sem_out = (so0, so1, so2)
    wid = lax.axis_index("s") * 2 + lax.axis_index("c")
    base = wid * PER_W
    iota = lax.iota(jnp.int32, LANES)

    pltpu.sync_copy(idx_hbm.at[pl.ds(base, PER_W)], pti_v)

    def prep(g, t):
        sl = pl.ds(g * LANES, LANES)
        pti_v[sl] = lax.bitwise_and(pti_v[sl], N_PAT - 1)
        spi_v[sl] = lax.rem(g * LANES + iota, HW)
        return t

    lax.fori_loop(0, PER_W // LANES, prep, 0)

    def issue_in(c, b):
        off = c * CHUNK
        pltpu.async_copy(sp_hbm.at[spi_v.at[pl.ds(off, CHUNK)]], sp_v.at[b],
                         sem_in[b])
        pltpu.async_copy(ppe_hbm.at[pti_v.at[pl.ds(off, CHUNK)]], pt_v.at[b],
                         sem_in[b])
        pltpu.async_copy(x_hbm.at[pl.ds(base + off, CHUNK)], x_v.at[b],
                         sem_in[b])

    def wait_in(b):
        pltpu.make_async_copy(sp_hbm.at[pl.ds(0, CHUNK)], sp_v.at[b],
                              sem_in[b]).wait()
        pltpu.make_async_copy(sp_hbm.at[pl.ds(0, CHUNK)], pt_v.at[b],
                              sem_in[b]).wait()
        pltpu.make_async_copy(x_hbm.at[pl.ds(0, CHUNK)], x_v.at[b],
                              sem_in[b]).wait()

    def issue_out(c, b):
        pltpu.async_copy(x_v.at[b], out_hbm.at[pl.ds(base + c * CHUNK, CHUNK)],
                         sem_out[b])

    def wait_out(b):
        pltpu.make_async_copy(x_v.at[b], out_hbm.at[pl.ds(0, CHUNK)],
                              sem_out[b]).wait()

    def adds(b):
        xb, spb, ptb = x_v.at[b], sp_v.at[b], pt_v.at[b]

        def add_body(j, t):
            for k in range(HALF // LANES):
                sl = pl.ds(k * LANES, LANES)
                sh = pl.ds(HALF + k * LANES, LANES)
                xb[j, sl] = xb[j, sl] + spb[j, sl]
                xb[j, sh] = xb[j, sh] + ptb[j, sl]
            return t

        lax.fori_loop(0, CHUNK, add_body, 0)

    issue_in(0, 0)
    issue_in(1, 1)
    issue_in(2, 2)
    wait_in(0); adds(0); issue_out(0, 0)
    wait_out(0); issue_in(3, 0)
    wait_in(1); adds(1); issue_out(1, 1)
    wait_out(1); issue_in(4, 1)
    wait_in(2); adds(2); issue_out(2, 2)

    def outer(co, t):
        for k in range(NBUF):
            c = NBUF * co + k
            pb = (k + 2) % NBUF

            @pl.when(c + 2 < NCHUNK)
            def _prefetch():
                wait_out(pb)
                issue_in(c + 2, pb)

            wait_in(k)
            adds(k)
            issue_out(c, k)
        return t

    lax.fori_loop(1, NCHUNK // NBUF, outer, 0)
    wait_out(0)
    wait_out(1)
    wait_out(2)


@jax.jit
def kernel(x, pattern_indices, spatial_pe, pattern_pe):
    b, h, w, d = x.shape
    xf = x.reshape(N, D_MODEL)
    idxf = pattern_indices.reshape(N).astype(jnp.int32)
    spf = spatial_pe.reshape(HW, HALF)

    mesh = plsc.VectorSubcoreMesh(core_axis_name="c", subcore_axis_name="s")
    out = pl.kernel(
        _body,
        out_type=jax.ShapeDtypeStruct((N, D_MODEL), jnp.float32),
        mesh=mesh,
        scratch_types=[
            pltpu.VMEM((NBUF, CHUNK, D_MODEL), jnp.float32),
            pltpu.VMEM((NBUF, CHUNK, HALF), jnp.float32),
            pltpu.VMEM((NBUF, CHUNK, HALF), jnp.float32),
            pltpu.VMEM((PER_W,), jnp.int32),
            pltpu.VMEM((PER_W,), jnp.int32),
            pltpu.SemaphoreType.DMA,
            pltpu.SemaphoreType.DMA,
            pltpu.SemaphoreType.DMA,
            pltpu.SemaphoreType.DMA,
            pltpu.SemaphoreType.DMA,
            pltpu.SemaphoreType.DMA,
        ],
    )(xf, idxf, spf, pattern_pe)
    return out.reshape(b, h, w, d)


# speedup vs baseline: 1.6662x; 1.4156x over previous
import jax
import jax.numpy as jnp
from jax import lax
from jax.experimental import pallas as pl
from jax.experimental.pallas import tpu as pltpu
from jax.experimental.pallas import tpu_sc as plsc

D_MODEL = 256
HALF = 128
N_PAT = 64
LANES = 16

B, H, W = 128, 30, 30
N = B * H * W
HW = H * W
NW = 32
PER_W = N // NW
CHUNK = 80
NCHUNK = PER_W // CHUNK
NBUF = 3
IMGS_PER_STEP = 2


def _gather_body(idx_hbm, ppe_hbm, out_hbm, pt_v, pti_v, si0, si1, si2,
                 so0, so1, so2):
    sem_in = (si0, si1, si2)
    sem_out = (so0, so1, so2)
    wid = lax.axis_index("s") * 2 + lax.axis_index("c")
    base = wid * PER_W

    pltpu.sync_copy(idx_hbm.at[pl.ds(base, PER_W)], pti_v)

    def prep(g, t):
        sl = pl.ds(g * LANES, LANES)
        pti_v[sl] = lax.bitwise_and(pti_v[sl], N_PAT - 1)
        return t

    lax.fori_loop(0, PER_W // LANES, prep, 0)

    def issue_in(c, b):
        pltpu.async_copy(ppe_hbm.at[pti_v.at[pl.ds(c * CHUNK, CHUNK)]],
                         pt_v.at[b], sem_in[b])

    def wait_in(b):
        pltpu.make_async_copy(out_hbm.at[pl.ds(0, CHUNK)], pt_v.at[b],
                              sem_in[b]).wait()

    def issue_out(c, b):
        pltpu.async_copy(pt_v.at[b], out_hbm.at[pl.ds(base + c * CHUNK, CHUNK)],
                         sem_out[b])

    def wait_out(b):
        pltpu.make_async_copy(pt_v.at[b], out_hbm.at[pl.ds(0, CHUNK)],
                              sem_out[b]).wait()

    issue_in(0, 0)
    issue_in(1, 1)
    issue_in(2, 2)
    wait_in(0); issue_out(0, 0)
    wait_out(0); issue_in(3, 0)
    wait_in(1); issue_out(1, 1)
    wait_out(1); issue_in(4, 1)
    wait_in(2); issue_out(2, 2)

    def outer(co, t):
        for k in range(NBUF):
            c = NBUF * co + k
            pb = (k + 2) % NBUF

            @pl.when(c + 2 < NCHUNK)
            def _prefetch():
                wait_out(pb)
                issue_in(c + 2, pb)

            wait_in(k)
            issue_out(c, k)
        return t

    lax.fori_loop(1, NCHUNK // NBUF, outer, 0)
    wait_out(0)
    wait_out(1)
    wait_out(2)


def _sc_gather(idxf, pattern_pe):
    mesh = plsc.VectorSubcoreMesh(core_axis_name="c", subcore_axis_name="s")
    return pl.kernel(
        _gather_body,
        out_type=jax.ShapeDtypeStruct((N, HALF), jnp.float32),
        mesh=mesh,
        scratch_types=[
            pltpu.VMEM((NBUF, CHUNK, HALF), jnp.float32),
            pltpu.VMEM((PER_W,), jnp.int32),
            pltpu.SemaphoreType.DMA,
            pltpu.SemaphoreType.DMA,
            pltpu.SemaphoreType.DMA,
            pltpu.SemaphoreType.DMA,
            pltpu.SemaphoreType.DMA,
            pltpu.SemaphoreType.DMA,
        ],
    )(idxf, pattern_pe)


def _add_body(x_ref, sp_ref, pc_ref, out_ref):
    xb = x_ref[...]
    pe = sp_ref[...]
    pc = pc_ref[...]
    lo = xb[..., :HALF] + pe[None, :, :, :]
    hi = xb[..., HALF:] + pc.reshape(IMGS_PER_STEP, H, W, HALF)
    out_ref[...] = jnp.concatenate([lo, hi], axis=-1)


def _tc_add(x, spatial_pe, penc):
    grid = (B // IMGS_PER_STEP,)
    return pl.pallas_call(
        _add_body,
        grid=grid,
        in_specs=[
            pl.BlockSpec((IMGS_PER_STEP, H, W, D_MODEL), lambda i: (i, 0, 0, 0)),
            pl.BlockSpec((H, W, HALF), lambda i: (0, 0, 0)),
            pl.BlockSpec((IMGS_PER_STEP * HW, HALF), lambda i: (i, 0)),
        ],
        out_specs=pl.BlockSpec((IMGS_PER_STEP, H, W, D_MODEL),
                               lambda i: (i, 0, 0, 0)),
        out_shape=jax.ShapeDtypeStruct((B, H, W, D_MODEL), jnp.float32),
    )(x, spatial_pe, penc)


@jax.jit
def kernel(x, pattern_indices, spatial_pe, pattern_pe):
    idxf = pattern_indices.reshape(N).astype(jnp.int32)
    penc = _sc_gather(idxf, pattern_pe)
    return _tc_add(x, spatial_pe, penc)
